# baseline (speedup 1.0000x reference)
import functools

import jax
import jax.numpy as jnp
from jax import lax
from jax.experimental import pallas as pl
from jax.experimental.pallas import tpu as pltpu
from jax.experimental.pallas import tpu_sc as plsc

VOCAB = 1000000
EMBED_DIM = 32
BATCH = 16384
HIST = 50

NC = 2
NS = 16
NW = NC * NS
BT = 128
NBT = BATCH // BT
BT_PER_W = NBT // NW
DT = EMBED_DIM // 8


YS = BT + 9


def _transpose_rows(rows_v, y_v, d_idx):
    for b0 in range(0, BT, 16):
        vecs = [(b, half, rows_v[b, pl.ds(half * 16, 16)])
                for b in range(b0, b0 + 16) for half in range(2)]
        for b, half, vec in vecs:
            plsc.store_scatter(y_v, [d_idx[half], jnp.full((16,), b)], vec)


def _body(idxt_hbm, tbl_hbm, out_hbm, idxt_v, rows_v, y_v, sem0, sem1):
    wid = lax.axis_index("s") * NC + lax.axis_index("c")
    iota = lax.iota(jnp.int32, 16)
    d_idx = (iota, iota + 16)
    sems = (sem0, sem1)

    def start(h, slot):
        pltpu.async_copy(tbl_hbm.at[idxt_v.at[h]],
                         rows_v.at[slot], sems[slot])

    def finish(bt, h, slot):
        pltpu.make_async_copy(tbl_hbm.at[pl.ds(0, BT)],
                              rows_v.at[slot], sems[slot]).wait()
        _transpose_rows(rows_v.at[slot], y_v.at[slot], d_idx)
        for dt in range(DT):
            pltpu.sync_copy(y_v.at[slot, pl.ds(dt * 8, 8), pl.ds(0, BT)],
                            out_hbm.at[h, dt, bt])

    def per_block(i, _):
        bt = wid * BT_PER_W + i
        pltpu.sync_copy(idxt_hbm.at[:, pl.ds(bt * BT, BT)], idxt_v)

        start(0, 0)

        def pair(p, _):
            h = 2 * p
            start(h + 1, 1)
            finish(bt, h, 0)
            start(h + 2, 0)
            finish(bt, h + 1, 1)
            return _

        lax.fori_loop(0, HIST // 2 - 1, pair, None)
        h = HIST - 2
        start(h + 1, 1)
        finish(bt, h, 0)
        finish(bt, h + 1, 1)
        return _

    lax.fori_loop(0, BT_PER_W, per_block, None)


@jax.jit
def _gather(action_ids, table):
    idxt = jnp.transpose(action_ids)
    mesh = plsc.VectorSubcoreMesh(core_axis_name="c", subcore_axis_name="s")
    k = functools.partial(
        pl.kernel,
        mesh=mesh,
        out_type=jax.ShapeDtypeStruct((HIST, DT, NBT, 8, BT), jnp.float32),
        scratch_types=[
            pltpu.VMEM((HIST, BT), jnp.int32),
            pltpu.VMEM((2, BT, EMBED_DIM), jnp.float32),
            pltpu.VMEM((2, EMBED_DIM, YS), jnp.float32),
            pltpu.SemaphoreType.DMA,
            pltpu.SemaphoreType.DMA,
        ],
        compiler_params=pltpu.CompilerParams(
            use_tc_tiling_on_sc=False, needs_layout_passes=False),
    )(_body)
    out5 = k(idxt, table)
    t = jnp.transpose(out5, (2, 4, 0, 1, 3))
    return t.reshape(BATCH, HIST, EMBED_DIM)


def kernel(action_ids, table):
    return _gather(action_ids, table)


# speedup vs baseline: 1.0070x; 1.0070x over previous
import functools

import jax
import jax.numpy as jnp
from jax import lax
from jax.experimental import pallas as pl
from jax.experimental.pallas import tpu as pltpu
from jax.experimental.pallas import tpu_sc as plsc

VOCAB = 1000000
EMBED_DIM = 32
BATCH = 16384
HIST = 50

NC = 2
NS = 16
NW = NC * NS
BT = 128
NBT = BATCH // BT
BT_PER_W = NBT // NW
DT = EMBED_DIM // 8


YS = BT + 9


def _transpose_rows(rows_v, y_v, d_idx):
    for b0 in range(0, BT, 8):
        vecs = [(b, half, rows_v[b, pl.ds(half * 16, 16)])
                for b in range(b0, b0 + 8) for half in range(2)]
        for b, half, vec in vecs:
            plsc.store_scatter(y_v, [d_idx[half], jnp.full((16,), b)], vec)


def _body(idxt_hbm, tbl_hbm, out_hbm, idxt_v, rows_v, y_v, sem0, sem1):
    wid = lax.axis_index("s") * NC + lax.axis_index("c")
    iota = lax.iota(jnp.int32, 16)
    d_idx = (iota, iota + 16)
    sems = (sem0, sem1)

    def start(h, slot):
        pltpu.async_copy(tbl_hbm.at[idxt_v.at[h]],
                         rows_v.at[slot], sems[slot])

    def finish(bt, h, slot):
        pltpu.make_async_copy(tbl_hbm.at[pl.ds(0, BT)],
                              rows_v.at[slot], sems[slot]).wait()
        _transpose_rows(rows_v.at[slot], y_v.at[slot], d_idx)
        for dt in range(DT):
            pltpu.sync_copy(y_v.at[slot, pl.ds(dt * 8, 8), pl.ds(0, BT)],
                            out_hbm.at[h, dt, bt])

    def per_block(i, _):
        bt = wid * BT_PER_W + i
        pltpu.sync_copy(idxt_hbm.at[:, pl.ds(bt * BT, BT)], idxt_v)

        start(0, 0)

        def pair(p, _):
            h = 2 * p
            start(h + 1, 1)
            finish(bt, h, 0)
            start(h + 2, 0)
            finish(bt, h + 1, 1)
            return _

        lax.fori_loop(0, HIST // 2 - 1, pair, None)
        h = HIST - 2
        start(h + 1, 1)
        finish(bt, h, 0)
        finish(bt, h + 1, 1)
        return _

    lax.fori_loop(0, BT_PER_W, per_block, None)


@jax.jit
def _gather(action_ids, table):
    idxt = jnp.transpose(action_ids)
    mesh = plsc.VectorSubcoreMesh(core_axis_name="c", subcore_axis_name="s")
    k = functools.partial(
        pl.kernel,
        mesh=mesh,
        out_type=jax.ShapeDtypeStruct((HIST, DT, NBT, 8, BT), jnp.float32),
        scratch_types=[
            pltpu.VMEM((HIST, BT), jnp.int32),
            pltpu.VMEM((2, BT, EMBED_DIM), jnp.float32),
            pltpu.VMEM((2, EMBED_DIM, YS), jnp.float32),
            pltpu.SemaphoreType.DMA,
            pltpu.SemaphoreType.DMA,
        ],
        compiler_params=pltpu.CompilerParams(
            use_tc_tiling_on_sc=False, needs_layout_passes=False),
    )(_body)
    out5 = k(idxt, table)
    t = jnp.transpose(out5, (2, 4, 0, 1, 3))
    return t.reshape(BATCH, HIST, EMBED_DIM)


def kernel(action_ids, table):
    return _gather(action_ids, table)
